# baseline (speedup 1.0000x reference)
import math

import jax
import jax.numpy as jnp
import numpy as np
from jax.experimental import pallas as pl

_BLOCK = 4096
_STRIPE = 16


def _pe_block_kernel(a0_ref, b0_ref, c1_ref, s1_ref, cl_ref, sl_ref, o_ref):
    a0 = a0_ref[0]
    b0 = b0_ref[0]
    cl = cl_ref[...]
    sl = sl_ref[...]
    for dh in range(_BLOCK // _STRIPE):
        c1 = c1_ref[pl.ds(dh, 1), :]
        s1 = s1_ref[pl.ds(dh, 1), :]
        a2 = a0 * c1 + b0 * s1
        b2 = b0 * c1 - a0 * s1
        o_ref[pl.ds(dh * _STRIPE, _STRIPE), :] = a2 * cl + b2 * sl


def _build_tables(seq_len, d_model, block, stripe):
    nblk = seq_len // block
    nstripe = block // stripe
    j = np.arange(d_model)
    w = np.exp(((j // 2) * 2).astype(np.float32) * (-(math.log(10000.0) / d_model)))
    even = (j % 2 == 0)[None, :]

    p0 = (np.arange(nblk, dtype=np.float32) * block)[:, None]
    ang0 = (p0 * w[None, :]).astype(np.float32)
    a0 = np.where(even, np.sin(ang0), np.cos(ang0)).astype(np.float32)
    b0 = np.where(even, np.cos(ang0), -np.sin(ang0)).astype(np.float32)

    dh = (np.arange(nstripe, dtype=np.float32) * stripe)[:, None]
    ang1 = (dh * w[None, :]).astype(np.float32)
    c1 = np.cos(ang1).astype(np.float32)
    s1 = np.sin(ang1).astype(np.float32)

    dl = np.arange(stripe, dtype=np.float32)[:, None]
    angl = (dl * w[None, :]).astype(np.float32)
    cl = np.cos(angl).astype(np.float32)
    sl = np.sin(angl).astype(np.float32)
    return a0[:, None, :], b0[:, None, :], c1, s1, cl, sl


def kernel(x, pe):
    seq_len = x.shape[1]
    d_model = pe.shape[2]
    block, stripe = _BLOCK, _STRIPE
    nblk = seq_len // block
    nstripe = block // stripe
    a0, b0, c1, s1, cl, sl = _build_tables(seq_len, d_model, block, stripe)

    out = pl.pallas_call(
        _pe_block_kernel,
        grid=(nblk,),
        in_specs=[
            pl.BlockSpec((1, 1, d_model), lambda i: (i, 0, 0)),
            pl.BlockSpec((1, 1, d_model), lambda i: (i, 0, 0)),
            pl.BlockSpec((nstripe, d_model), lambda i: (0, 0)),
            pl.BlockSpec((nstripe, d_model), lambda i: (0, 0)),
            pl.BlockSpec((stripe, d_model), lambda i: (0, 0)),
            pl.BlockSpec((stripe, d_model), lambda i: (0, 0)),
        ],
        out_specs=pl.BlockSpec((block, d_model), lambda i: (i, 0)),
        out_shape=jax.ShapeDtypeStruct((seq_len, d_model), jnp.float32),
    )(a0, b0, c1, s1, cl, sl)
    return out[None]


# speedup vs baseline: 1.2729x; 1.2729x over previous
import math

import jax
import jax.numpy as jnp
import numpy as np
from jax.experimental import pallas as pl
from jax.experimental.pallas import tpu as pltpu

_BLOCK = 512
_STRIPE = 16
_NBUF = 4
_SEQ_LEN = 8192


def _build_tables(seq_len, d_model, block, stripe):
    nblk = seq_len // block
    nstripe = block // stripe
    j = np.arange(d_model)
    w = np.exp(((j // 2) * 2).astype(np.float32) * (-(math.log(10000.0) / d_model)))
    even = (j % 2 == 0)[None, :]

    p0 = (np.arange(nblk, dtype=np.float32) * block)[:, None]
    ang0 = (p0 * w[None, :]).astype(np.float32)
    a0 = np.where(even, np.sin(ang0), np.cos(ang0)).astype(np.float32)
    b0 = np.where(even, np.cos(ang0), -np.sin(ang0)).astype(np.float32)

    dh = (np.arange(nstripe, dtype=np.float32) * stripe)[:, None]
    ang1 = (dh * w[None, :]).astype(np.float32)
    c1 = np.cos(ang1).astype(np.float32)
    s1 = np.sin(ang1).astype(np.float32)

    dl = np.arange(stripe, dtype=np.float32)[:, None]
    angl = (dl * w[None, :]).astype(np.float32)
    cl = np.cos(angl).astype(np.float32)
    sl = np.sin(angl).astype(np.float32)
    return a0, b0, c1, s1, cl, sl


def _ring_kernel(a0_ref, b0_ref, c1_ref, s1_ref, cl_ref, sl_ref, o_hbm,
                 bufs, sems):
    nblk = _SEQ_LEN // _BLOCK
    nstripe = _BLOCK // _STRIPE
    cl = cl_ref[...]
    sl = sl_ref[...]

    def compute_block(i, slot):
        a0 = a0_ref[pl.ds(i, 1), :]
        b0 = b0_ref[pl.ds(i, 1), :]
        for dh in range(nstripe):
            c1 = c1_ref[pl.ds(dh, 1), :]
            s1 = s1_ref[pl.ds(dh, 1), :]
            a2 = a0 * c1 + b0 * s1
            b2 = b0 * c1 - a0 * s1
            bufs[slot, pl.ds(dh * _STRIPE, _STRIPE), :] = a2 * cl + b2 * sl

    for i in range(nblk):
        slot = i % _NBUF
        if i >= _NBUF:
            pltpu.make_async_copy(
                bufs.at[slot], o_hbm.at[pl.ds((i - _NBUF) * _BLOCK, _BLOCK)],
                sems.at[slot]).wait()
        compute_block(i, slot)
        pltpu.make_async_copy(
            bufs.at[slot], o_hbm.at[pl.ds(i * _BLOCK, _BLOCK)],
            sems.at[slot]).start()
    for k in range(_NBUF):
        i = nblk - _NBUF + k
        slot = i % _NBUF
        pltpu.make_async_copy(
            bufs.at[slot], o_hbm.at[pl.ds(i * _BLOCK, _BLOCK)],
            sems.at[slot]).wait()


def kernel(x, pe):
    seq_len = x.shape[1]
    d_model = pe.shape[2]
    block, stripe = _BLOCK, _STRIPE
    nstripe = block // stripe
    a0, b0, c1, s1, cl, sl = _build_tables(seq_len, d_model, block, stripe)

    out = pl.pallas_call(
        _ring_kernel,
        in_specs=[
            pl.BlockSpec(memory_space=pltpu.VMEM),
            pl.BlockSpec(memory_space=pltpu.VMEM),
            pl.BlockSpec(memory_space=pltpu.VMEM),
            pl.BlockSpec(memory_space=pltpu.VMEM),
            pl.BlockSpec(memory_space=pltpu.VMEM),
            pl.BlockSpec(memory_space=pltpu.VMEM),
        ],
        out_specs=pl.BlockSpec(memory_space=pl.ANY),
        out_shape=jax.ShapeDtypeStruct((seq_len, d_model), jnp.float32),
        scratch_shapes=[
            pltpu.VMEM((_NBUF, block, d_model), jnp.float32),
            pltpu.SemaphoreType.DMA((_NBUF,)),
        ],
    )(a0, b0, c1, s1, cl, sl)
    return out[None]
